# Initial kernel scaffold; baseline (speedup 1.0000x reference)
import jax
import jax.numpy as jnp
from jax.experimental import pallas as pl


def kernel(x, adj, W, a_src, a_dst):
    raise NotImplementedError("write your pallas kernel here")


# speedup vs baseline: 1.3362x; 1.3362x over previous
import jax
import jax.numpy as jnp
from jax.experimental import pallas as pl

SLOPE = 0.2
HEADS = 4
NEG = -9e15

BR = 256
BO = 512


def _proj_body(x_ref, w_ref, asrc_ref, adst_ref, h_ref, f1_ref, f2_ref):
    h = jnp.dot(x_ref[...], w_ref[0], preferred_element_type=jnp.float32)
    h_ref[0] = h
    nt = (((1,), (1,)), ((), ()))
    f1_ref[0] = jax.lax.dot_general(h, asrc_ref[0], nt,
                                    preferred_element_type=jnp.float32)
    f2_ref[0] = jax.lax.dot_general(adst_ref[0], h, nt,
                                    preferred_element_type=jnp.float32)


def _layer_body(adj_ref, h_ref, f1_ref, f2_ref, x_ref, o_ref):
    hid = pl.program_id(1)
    e = f1_ref[0] + f2_ref[0]
    e = jnp.where(e >= 0.0, e, e * SLOPE)
    e = jnp.where(adj_ref[...] > 0.0, e, NEG)
    m = jnp.max(e, axis=1, keepdims=True)
    p = jnp.exp(e - m)
    s = jnp.sum(p, axis=1, keepdims=True)
    out = jnp.dot(p / s, h_ref[0], preferred_element_type=jnp.float32)
    out = out + x_ref[...]
    out = jnp.where(out > 0.0, out, jnp.exp(out) - 1.0)
    out = out * (1.0 / HEADS)

    @pl.when(hid == 0)
    def _():
        o_ref[...] = out

    @pl.when(hid != 0)
    def _():
        o_ref[...] = o_ref[...] + out


def _outer_body(a_ref, b_ref, o_ref):
    nt = (((1,), (1,)), ((), ()))
    o_ref[...] = jax.lax.dot_general(a_ref[...], b_ref[...], nt,
                                     preferred_element_type=jnp.float32)


def _gat_layer(xin, adj, W, a_src2, a_dst2, interpret=False):
    N, D = xin.shape
    h_all, f1, f2 = pl.pallas_call(
        _proj_body,
        grid=(HEADS,),
        in_specs=[
            pl.BlockSpec((N, D), lambda h: (0, 0)),
            pl.BlockSpec((1, D, D), lambda h: (h, 0, 0)),
            pl.BlockSpec((1, 1, D), lambda h: (h, 0, 0)),
            pl.BlockSpec((1, 1, D), lambda h: (h, 0, 0)),
        ],
        out_specs=[
            pl.BlockSpec((1, N, D), lambda h: (h, 0, 0)),
            pl.BlockSpec((1, N, 1), lambda h: (h, 0, 0)),
            pl.BlockSpec((1, 1, N), lambda h: (h, 0, 0)),
        ],
        out_shape=[
            jax.ShapeDtypeStruct((HEADS, N, D), jnp.float32),
            jax.ShapeDtypeStruct((HEADS, N, 1), jnp.float32),
            jax.ShapeDtypeStruct((HEADS, 1, N), jnp.float32),
        ],
        interpret=interpret,
    )(xin, W, a_src2, a_dst2)

    nr = N // BR
    out = pl.pallas_call(
        _layer_body,
        grid=(nr, HEADS),
        in_specs=[
            pl.BlockSpec((BR, N), lambda r, h: (r, 0)),
            pl.BlockSpec((1, N, D), lambda r, h: (h, 0, 0)),
            pl.BlockSpec((1, BR, 1), lambda r, h: (h, r, 0)),
            pl.BlockSpec((1, 1, N), lambda r, h: (h, 0, 0)),
            pl.BlockSpec((BR, D), lambda r, h: (r, 0)),
        ],
        out_specs=pl.BlockSpec((BR, D), lambda r, h: (r, 0)),
        out_shape=jax.ShapeDtypeStruct((N, D), jnp.float32),
        interpret=interpret,
    )(adj, h_all, f1, f2, xin)
    return out


def kernel(x, adj, W, a_src, a_dst, interpret=False):
    N, D = x.shape
    a_src2 = a_src[:, None, :]
    a_dst2 = a_dst[:, None, :]

    m = _gat_layer(x, adj, W, a_src2, a_dst2, interpret)
    m = _gat_layer(m, adj, W, a_src2, a_dst2, interpret)
    m = _gat_layer(m, adj, W, a_src2, a_dst2, interpret)

    nb = N // BO
    ret = pl.pallas_call(
        _outer_body,
        grid=(nb, nb),
        in_specs=[
            pl.BlockSpec((BO, D), lambda i, j: (i, 0)),
            pl.BlockSpec((BO, D), lambda i, j: (j, 0)),
        ],
        out_specs=pl.BlockSpec((BO, BO), lambda i, j: (i, j)),
        out_shape=jax.ShapeDtypeStruct((N, N), jnp.float32),
        interpret=interpret,
    )(m, m)
    return ret


# speedup vs baseline: 1.5429x; 1.1547x over previous
import jax
import jax.numpy as jnp
from jax.experimental import pallas as pl
from jax.experimental.pallas import tpu as pltpu

SLOPE = 0.2
HEADS = 4
NEG = -9e15

BR = 512
BO = 512

_NT = (((1,), (1,)), ((), ()))


def _layer_body(adj_ref, x_ref, w_ref, asrc_ref, adst_ref, o_ref,
                h_scr, f1_scr, f2_scr):
    r = pl.program_id(0)
    hid = pl.program_id(1)

    @pl.when(r == 0)
    def _():
        h = jnp.dot(x_ref[...], w_ref[0], preferred_element_type=jnp.float32)
        h_scr[hid] = h
        f1_scr[hid] = jax.lax.dot_general(
            h, asrc_ref[0], _NT, preferred_element_type=jnp.float32)
        f2_scr[hid] = jax.lax.dot_general(
            adst_ref[0], h, _NT, preferred_element_type=jnp.float32)

    rows = pl.ds(r * BR, BR)
    e = f1_scr[hid, rows, :] + f2_scr[hid]
    e = jnp.maximum(e, e * SLOPE)
    e = jnp.where(adj_ref[...] > 0.0, e, NEG)
    m = jnp.max(e, axis=1, keepdims=True)
    p = jnp.exp(e - m)
    s = jnp.sum(p, axis=1, keepdims=True)
    out = jnp.dot(p, h_scr[hid], preferred_element_type=jnp.float32)
    out = out / s + x_ref[rows, :]
    out = jnp.where(out > 0.0, out, jnp.exp(out) - 1.0)
    out = out * (1.0 / HEADS)

    @pl.when(hid == 0)
    def _():
        o_ref[...] = out

    @pl.when(hid != 0)
    def _():
        o_ref[...] = o_ref[...] + out


def _outer_body(a_ref, b_ref, o_ref):
    o_ref[...] = jax.lax.dot_general(a_ref[...], b_ref[...], _NT,
                                     preferred_element_type=jnp.float32)


def _gat_layer(xin, adj, W, a_src2, a_dst2, interpret=False):
    N, D = xin.shape
    nr = N // BR
    return pl.pallas_call(
        _layer_body,
        grid=(nr, HEADS),
        in_specs=[
            pl.BlockSpec((BR, N), lambda r, h: (r, 0)),
            pl.BlockSpec((N, D), lambda r, h: (0, 0)),
            pl.BlockSpec((1, D, D), lambda r, h: (h, 0, 0)),
            pl.BlockSpec((1, 1, D), lambda r, h: (h, 0, 0)),
            pl.BlockSpec((1, 1, D), lambda r, h: (h, 0, 0)),
        ],
        out_specs=pl.BlockSpec((BR, D), lambda r, h: (r, 0)),
        out_shape=jax.ShapeDtypeStruct((N, D), jnp.float32),
        scratch_shapes=[
            pltpu.VMEM((HEADS, N, D), jnp.float32),
            pltpu.VMEM((HEADS, N, 1), jnp.float32),
            pltpu.VMEM((HEADS, 1, N), jnp.float32),
        ],
        interpret=interpret,
    )(adj, xin, W, a_src2, a_dst2)


def kernel(x, adj, W, a_src, a_dst, interpret=False):
    N, D = x.shape
    a_src2 = a_src[:, None, :]
    a_dst2 = a_dst[:, None, :]

    m = _gat_layer(x, adj, W, a_src2, a_dst2, interpret)
    m = _gat_layer(m, adj, W, a_src2, a_dst2, interpret)
    m = _gat_layer(m, adj, W, a_src2, a_dst2, interpret)

    nb = N // BO
    ret = pl.pallas_call(
        _outer_body,
        grid=(nb, nb),
        in_specs=[
            pl.BlockSpec((BO, D), lambda i, j: (i, 0)),
            pl.BlockSpec((BO, D), lambda i, j: (j, 0)),
        ],
        out_specs=pl.BlockSpec((BO, BO), lambda i, j: (i, j)),
        out_shape=jax.ShapeDtypeStruct((N, N), jnp.float32),
        interpret=interpret,
    )(m, m)
    return ret
